# baseline (speedup 1.0000x reference)
import functools

import jax
import jax.numpy as jnp
from jax import lax
from jax.experimental import pallas as pl
from jax.experimental.pallas import tpu as pltpu
from jax.experimental.pallas import tpu_sc as plsc

VOCAB = 100000
DIM = 64
BATCH = 16384
NC = 2
NS = 16
NW = NC * NS
BPW = BATCH // NW
CHUNK = 128
NCHUNK = BPW // CHUNK
L = 16

CB = 2048
HB = CB // 2
RGRID = (VOCAB + CB - 1) // CB
P2ROWS = RGRID * HB


def _tc_repack01_body(l0, l1, o01):
    o01[...] = jnp.transpose(jnp.concatenate([l0[...], l1[...]], axis=0))


def _tc_repack2_body(l2, o2):
    o2[...] = jnp.transpose(jnp.concatenate(
        [l2[:, 0:HB], l2[:, HB:CB]], axis=0))


def _tc_repack01(lt0, lt1):
    inblk = pl.BlockSpec((DIM, CB), lambda i: (0, i))
    return pl.pallas_call(
        _tc_repack01_body,
        grid=(RGRID,),
        in_specs=[inblk, inblk],
        out_specs=pl.BlockSpec((CB, 2 * DIM), lambda i: (i, 0)),
        out_shape=jax.ShapeDtypeStruct((VOCAB, 2 * DIM), jnp.float32),
    )(lt0, lt1)


def _tc_repack2(lt2):
    return pl.pallas_call(
        _tc_repack2_body,
        grid=(RGRID,),
        in_specs=[pl.BlockSpec((DIM, CB), lambda i: (0, i))],
        out_specs=pl.BlockSpec((HB, 2 * DIM), lambda i: (i, 0)),
        out_shape=jax.ShapeDtypeStruct((P2ROWS, 2 * DIM), jnp.float32),
    )(lt2)


def _sc_pipeline(jobs, base, bufs, gs, ws):
    gats = [None, None]
    wbs = [None, None]

    def issue_wb(kk):
        idx_, src_, dst_, col_, full_, j_ = jobs[kk]
        s_ = kk % 2
        gats[s_].wait()
        rows = pl.ds(base + j_ * CHUNK, CHUNK)
        if full_:
            wbs[s_] = pltpu.async_copy(bufs[s_], dst_.at[rows], ws[s_])
        else:
            wbs[s_] = pltpu.async_copy(
                bufs[s_].at[:, pl.ds(col_, DIM)],
                dst_.at[rows, pl.ds(col_, DIM)], ws[s_])

    for k, (idx, src, dst, col, full, j) in enumerate(jobs):
        s = k % 2
        if wbs[s] is not None:
            wbs[s].wait()
        gats[s] = pltpu.async_copy(
            src.at[idx.at[pl.ds(j * CHUNK, CHUNK)]], bufs[s], gs[s])
        if k >= 1:
            issue_wb(k - 1)
    issue_wb(len(jobs) - 1)
    wbs[0].wait()
    wbs[1].wait()


def _sc_gather01_body(xT, p01, e01, idx0, idx1, bufA, bufB, gsA, gsB,
                      wsA, wsB):
    wid = lax.axis_index("s") * NC + lax.axis_index("c")
    base = wid * BPW
    for t, idx in enumerate((idx0, idx1)):
        pltpu.sync_copy(xT.at[t, pl.ds(base, BPW)], idx)
    jobs = []
    for idx, col in ((idx0, 0), (idx1, DIM)):
        for j in range(NCHUNK):
            jobs.append((idx, p01, e01, col, False, j))
    _sc_pipeline(jobs, base, (bufA, bufB), (gsA, gsB), (wsA, wsB))


def _sc_gather2_body(xT, p2, e2, idx2, bufA, bufB, gsA, gsB, wsA, wsB):
    wid = lax.axis_index("s") * NC + lax.axis_index("c")
    base = wid * BPW
    pltpu.sync_copy(xT.at[2, pl.ds(base, BPW)], idx2)
    for i in range(BPW // L):
        v = idx2[pl.ds(i * L, L)]
        idx2[pl.ds(i * L, L)] = ((v >> 11) << 10) | (v & 1023)
    jobs = [(idx2, p2, e2, 0, True, j) for j in range(NCHUNK)]
    _sc_pipeline(jobs, base, (bufA, bufB), (gsA, gsB), (wsA, wsB))


_SC_SCRATCH_COMMON = [
    pltpu.VMEM((CHUNK, 2 * DIM), jnp.float32),
    pltpu.VMEM((CHUNK, 2 * DIM), jnp.float32),
    pltpu.SemaphoreType.DMA,
    pltpu.SemaphoreType.DMA,
    pltpu.SemaphoreType.DMA,
    pltpu.SemaphoreType.DMA,
]

_EPACKED = jax.ShapeDtypeStruct((BATCH, 2 * DIM), jnp.float32)


@functools.cache
def _sc_gather01():
    return pl.kernel(
        _sc_gather01_body,
        mesh=plsc.VectorSubcoreMesh(core_axis_name="c", subcore_axis_name="s"),
        out_type=_EPACKED,
        scratch_types=[
            pltpu.VMEM((BPW,), jnp.int32),
            pltpu.VMEM((BPW,), jnp.int32),
        ] + _SC_SCRATCH_COMMON,
        compiler_params=pltpu.CompilerParams(use_tc_tiling_on_sc=False),
    )


@functools.cache
def _sc_gather2():
    return pl.kernel(
        _sc_gather2_body,
        mesh=plsc.VectorSubcoreMesh(core_axis_name="c", subcore_axis_name="s"),
        out_type=_EPACKED,
        scratch_types=[
            pltpu.VMEM((BPW,), jnp.int32),
        ] + _SC_SCRATCH_COMMON,
        compiler_params=pltpu.CompilerParams(use_tc_tiling_on_sc=False),
    )


BM = 2048


def _tc_proj_body(xb, e01, e2p, w, b, o):
    h = ((xb[2, 0, :] >> 10) & 1).reshape(BM, 1)
    e2 = jnp.where(h > 0, e2p[:, DIM:2 * DIM], e2p[:, 0:DIM])
    acc = lax.dot_general(w[0:2 * DIM, :], e01[...],
                          (((0,), (1,)), ((), ())),
                          preferred_element_type=jnp.float32)
    acc += lax.dot_general(w[2 * DIM:3 * DIM, :], e2,
                           (((0,), (1,)), ((), ())),
                           preferred_element_type=jnp.float32)
    o[...] = acc * 8.0 + b[...]


def _tc_project(xT3, e01, e2p, w, bcol):
    grid = (BATCH // BM,)
    eblk = pl.BlockSpec((BM, 2 * DIM), lambda i: (i, 0))
    outT = pl.pallas_call(
        _tc_proj_body,
        grid=grid,
        in_specs=[
            pl.BlockSpec((3, 1, BM), lambda i: (0, 0, i)),
            eblk, eblk,
            pl.BlockSpec((3 * DIM, DIM), lambda i: (0, 0)),
            pl.BlockSpec((DIM, 1), lambda i: (0, 0)),
        ],
        out_specs=pl.BlockSpec((DIM, BM), lambda i: (0, i)),
        out_shape=jax.ShapeDtypeStruct((DIM, BATCH), jnp.float32),
    )(xT3, e01, e2p, w, bcol)
    return outT.T


def kernel(x, lut0, lut1, lut2, W, b):
    xT = x.astype(jnp.int32).T
    p01 = _tc_repack01(lut0.T, lut1.T)
    p2 = _tc_repack2(lut2.T)
    e01 = _sc_gather01()(xT, p01)
    e2p = _sc_gather2()(xT, p2)
    return _tc_project(xT.reshape(3, 1, BATCH), e01, e2p, W,
                       b.reshape(DIM, 1))


# speedup vs baseline: 1.3383x; 1.3383x over previous
import functools

import jax
import jax.numpy as jnp
from jax import lax
from jax.experimental import pallas as pl
from jax.experimental.pallas import tpu as pltpu
from jax.experimental.pallas import tpu_sc as plsc

VOCAB = 100000
DIM = 64
BATCH = 16384
NC = 2
NS = 16
NW = NC * NS
BPW = BATCH // NW
CHUNK = 128
NCHUNK = BPW // CHUNK
L = 16

CB = 4096
HB = CB // 2
RGRID = (VOCAB + CB - 1) // CB
P2ROWS = RGRID * HB
CSH = CB.bit_length() - 1
HSH = CSH - 1


def _tc_repack_body(l0, l1, l2, o01, o2):
    o01[...] = jnp.transpose(jnp.concatenate([l0[...], l1[...]], axis=0))
    o2[...] = jnp.transpose(jnp.concatenate(
        [l2[:, 0:HB], l2[:, HB:CB]], axis=0))


def _tc_repack(lt0, lt1, lt2):
    inblk = pl.BlockSpec((DIM, CB), lambda i: (0, i))
    return pl.pallas_call(
        _tc_repack_body,
        grid=(RGRID,),
        in_specs=[inblk, inblk, inblk],
        out_specs=[pl.BlockSpec((CB, 2 * DIM), lambda i: (i, 0)),
                   pl.BlockSpec((HB, 2 * DIM), lambda i: (i, 0))],
        out_shape=(jax.ShapeDtypeStruct((VOCAB, 2 * DIM), jnp.float32),
                   jax.ShapeDtypeStruct((P2ROWS, 2 * DIM), jnp.float32)),
    )(lt0, lt1, lt2)


def _sc_pipeline(jobs, base, bufs, gs, ws):
    gats = [None, None]
    wbs = [None, None]

    def issue_wb(kk):
        idx_, src_, dst_, col_, full_, j_ = jobs[kk]
        s_ = kk % 2
        gats[s_].wait()
        rows = pl.ds(base + j_ * CHUNK, CHUNK)
        if full_:
            wbs[s_] = pltpu.async_copy(bufs[s_], dst_.at[rows], ws[s_])
        else:
            wbs[s_] = pltpu.async_copy(
                bufs[s_].at[:, pl.ds(col_, DIM)],
                dst_.at[rows, pl.ds(col_, DIM)], ws[s_])

    for k, (idx, src, dst, col, full, j) in enumerate(jobs):
        s = k % 2
        if wbs[s] is not None:
            wbs[s].wait()
        gats[s] = pltpu.async_copy(
            src.at[idx.at[pl.ds(j * CHUNK, CHUNK)]], bufs[s], gs[s])
        if k >= 1:
            issue_wb(k - 1)
    issue_wb(len(jobs) - 1)
    wbs[0].wait()
    wbs[1].wait()


def _sc_gather_body(xT, p01, p2, e01, e2, idx0, idx1, idx2, bufA, bufB,
                    gsA, gsB, wsA, wsB):
    wid = lax.axis_index("s") * NC + lax.axis_index("c")
    base = wid * BPW
    for t, idx in enumerate((idx0, idx1, idx2)):
        pltpu.sync_copy(xT.at[t, pl.ds(base, BPW)], idx)
    for i in range(BPW // L):
        v = idx2[pl.ds(i * L, L)]
        idx2[pl.ds(i * L, L)] = ((v >> CSH) << HSH) | (v & (HB - 1))
    jobs = []
    for idx, src, dst, col, full in ((idx0, p01, e01, 0, False),
                                     (idx1, p01, e01, DIM, False),
                                     (idx2, p2, e2, 0, True)):
        for j in range(NCHUNK):
            jobs.append((idx, src, dst, col, full, j))
    _sc_pipeline(jobs, base, (bufA, bufB), (gsA, gsB), (wsA, wsB))


@functools.cache
def _sc_gather():
    packed = jax.ShapeDtypeStruct((BATCH, 2 * DIM), jnp.float32)
    return pl.kernel(
        _sc_gather_body,
        mesh=plsc.VectorSubcoreMesh(core_axis_name="c", subcore_axis_name="s"),
        out_type=(packed, packed),
        scratch_types=[
            pltpu.VMEM((BPW,), jnp.int32),
            pltpu.VMEM((BPW,), jnp.int32),
            pltpu.VMEM((BPW,), jnp.int32),
            pltpu.VMEM((CHUNK, 2 * DIM), jnp.float32),
            pltpu.VMEM((CHUNK, 2 * DIM), jnp.float32),
            pltpu.SemaphoreType.DMA,
            pltpu.SemaphoreType.DMA,
            pltpu.SemaphoreType.DMA,
            pltpu.SemaphoreType.DMA,
        ],
        compiler_params=pltpu.CompilerParams(use_tc_tiling_on_sc=False),
    )


BM = 2048


def _tc_proj_body(xb, e01, e2p, w, b, o):
    h = ((xb[2, 0, :] >> HSH) & 1).reshape(BM, 1)
    e2 = jnp.where(h > 0, e2p[:, DIM:2 * DIM], e2p[:, 0:DIM])
    acc = lax.dot_general(w[0:2 * DIM, :], e01[...],
                          (((0,), (1,)), ((), ())),
                          preferred_element_type=jnp.float32)
    acc += lax.dot_general(w[2 * DIM:3 * DIM, :], e2,
                           (((0,), (1,)), ((), ())),
                           preferred_element_type=jnp.float32)
    o[...] = acc * 8.0 + b[...]


def _tc_project(xT3, e01, e2p, w, bcol):
    grid = (BATCH // BM,)
    eblk = pl.BlockSpec((BM, 2 * DIM), lambda i: (i, 0))
    outT = pl.pallas_call(
        _tc_proj_body,
        grid=grid,
        in_specs=[
            pl.BlockSpec((3, 1, BM), lambda i: (0, 0, i)),
            eblk, eblk,
            pl.BlockSpec((3 * DIM, DIM), lambda i: (0, 0)),
            pl.BlockSpec((DIM, 1), lambda i: (0, 0)),
        ],
        out_specs=pl.BlockSpec((DIM, BM), lambda i: (0, i)),
        out_shape=jax.ShapeDtypeStruct((DIM, BATCH), jnp.float32),
    )(xT3, e01, e2p, w, bcol)
    return outT.T


def kernel(x, lut0, lut1, lut2, W, b):
    xT = x.astype(jnp.int32).T
    p01, p2 = _tc_repack(lut0.T, lut1.T, lut2.T)
    e01, e2p = _sc_gather()(xT, p01, p2)
    return _tc_project(xT.reshape(3, 1, BATCH), e01, e2p, W,
                       b.reshape(DIM, 1))


# speedup vs baseline: 1.3745x; 1.0270x over previous
import functools

import jax
import jax.numpy as jnp
from jax import lax
from jax.experimental import pallas as pl
from jax.experimental.pallas import tpu as pltpu
from jax.experimental.pallas import tpu_sc as plsc

VOCAB = 100000
DIM = 64
BATCH = 16384
NC = 2
NS = 16
NW = NC * NS
BPW = BATCH // NW
CHUNK = 128
NCHUNK = BPW // CHUNK
L = 16

CB = 8192
HB = CB // 2
RGRID = (VOCAB + CB - 1) // CB
P2ROWS = RGRID * HB
CSH = CB.bit_length() - 1
HSH = CSH - 1


def _tc_repack_body(l0, l1, l2, o01, o2):
    o01[...] = jnp.transpose(jnp.concatenate([l0[...], l1[...]], axis=0))
    o2[...] = jnp.transpose(jnp.concatenate(
        [l2[:, 0:HB], l2[:, HB:CB]], axis=0))


def _tc_repack(lt0, lt1, lt2):
    inblk = pl.BlockSpec((DIM, CB), lambda i: (0, i))
    return pl.pallas_call(
        _tc_repack_body,
        grid=(RGRID,),
        in_specs=[inblk, inblk, inblk],
        out_specs=[pl.BlockSpec((CB, 2 * DIM), lambda i: (i, 0)),
                   pl.BlockSpec((HB, 2 * DIM), lambda i: (i, 0))],
        out_shape=(jax.ShapeDtypeStruct((VOCAB, 2 * DIM), jnp.float32),
                   jax.ShapeDtypeStruct((P2ROWS, 2 * DIM), jnp.float32)),
    )(lt0, lt1, lt2)


def _sc_pipeline(jobs, base, bufs, gs, ws):
    gats = [None, None]
    wbs = [None, None]

    def issue_wb(kk):
        idx_, src_, dst_, col_, full_, j_ = jobs[kk]
        s_ = kk % 2
        gats[s_].wait()
        rows = pl.ds(base + j_ * CHUNK, CHUNK)
        if full_:
            wbs[s_] = pltpu.async_copy(bufs[s_], dst_.at[rows], ws[s_])
        else:
            wbs[s_] = pltpu.async_copy(
                bufs[s_].at[:, pl.ds(col_, DIM)],
                dst_.at[rows, pl.ds(col_, DIM)], ws[s_])

    for k, (idx, src, dst, col, full, j) in enumerate(jobs):
        s = k % 2
        if wbs[s] is not None:
            wbs[s].wait()
        gats[s] = pltpu.async_copy(
            src.at[idx.at[pl.ds(j * CHUNK, CHUNK)]], bufs[s], gs[s])
        if k >= 1:
            issue_wb(k - 1)
    issue_wb(len(jobs) - 1)
    wbs[0].wait()
    wbs[1].wait()


def _sc_gather_body(xT, p01, p2, e01, e2, idx0, idx1, idx2, bufA, bufB,
                    gsA, gsB, wsA, wsB):
    wid = lax.axis_index("s") * NC + lax.axis_index("c")
    base = wid * BPW
    for t, idx in enumerate((idx0, idx1, idx2)):
        pltpu.sync_copy(xT.at[t, pl.ds(base, BPW)], idx)
    for i in range(BPW // L):
        v = idx2[pl.ds(i * L, L)]
        idx2[pl.ds(i * L, L)] = ((v >> CSH) << HSH) | (v & (HB - 1))
    jobs = []
    for idx, src, dst, col, full in ((idx0, p01, e01, 0, False),
                                     (idx1, p01, e01, DIM, False),
                                     (idx2, p2, e2, 0, True)):
        for j in range(NCHUNK):
            jobs.append((idx, src, dst, col, full, j))
    _sc_pipeline(jobs, base, (bufA, bufB), (gsA, gsB), (wsA, wsB))


@functools.cache
def _sc_gather():
    packed = jax.ShapeDtypeStruct((BATCH, 2 * DIM), jnp.float32)
    return pl.kernel(
        _sc_gather_body,
        mesh=plsc.VectorSubcoreMesh(core_axis_name="c", subcore_axis_name="s"),
        out_type=(packed, packed),
        scratch_types=[
            pltpu.VMEM((BPW,), jnp.int32),
            pltpu.VMEM((BPW,), jnp.int32),
            pltpu.VMEM((BPW,), jnp.int32),
            pltpu.VMEM((CHUNK, 2 * DIM), jnp.float32),
            pltpu.VMEM((CHUNK, 2 * DIM), jnp.float32),
            pltpu.SemaphoreType.DMA,
            pltpu.SemaphoreType.DMA,
            pltpu.SemaphoreType.DMA,
            pltpu.SemaphoreType.DMA,
        ],
        compiler_params=pltpu.CompilerParams(use_tc_tiling_on_sc=False),
    )


BM = 2048


def _tc_proj_body(xb, e01, e2p, w, b, o):
    h = ((xb[2, 0, :] >> HSH) & 1).reshape(BM, 1)
    e2 = jnp.where(h > 0, e2p[:, DIM:2 * DIM], e2p[:, 0:DIM])
    acc = lax.dot_general(w[0:2 * DIM, :], e01[...],
                          (((0,), (1,)), ((), ())),
                          preferred_element_type=jnp.float32)
    acc += lax.dot_general(w[2 * DIM:3 * DIM, :], e2,
                           (((0,), (1,)), ((), ())),
                           preferred_element_type=jnp.float32)
    o[...] = acc * 8.0 + b[...]


def _tc_project(xT3, e01, e2p, w, bcol):
    grid = (BATCH // BM,)
    eblk = pl.BlockSpec((BM, 2 * DIM), lambda i: (i, 0))
    outT = pl.pallas_call(
        _tc_proj_body,
        grid=grid,
        in_specs=[
            pl.BlockSpec((3, 1, BM), lambda i: (0, 0, i)),
            eblk, eblk,
            pl.BlockSpec((3 * DIM, DIM), lambda i: (0, 0)),
            pl.BlockSpec((DIM, 1), lambda i: (0, 0)),
        ],
        out_specs=pl.BlockSpec((DIM, BM), lambda i: (0, i)),
        out_shape=jax.ShapeDtypeStruct((DIM, BATCH), jnp.float32),
    )(xT3, e01, e2p, w, bcol)
    return outT.T


def kernel(x, lut0, lut1, lut2, W, b):
    xT = x.astype(jnp.int32).T
    p01, p2 = _tc_repack(lut0.T, lut1.T, lut2.T)
    e01, e2p = _sc_gather()(xT, p01, p2)
    return _tc_project(xT.reshape(3, 1, BATCH), e01, e2p, W,
                       b.reshape(DIM, 1))
